# baseline (speedup 1.0000x reference)
import functools

import jax
import jax.numpy as jnp
from jax import lax
from jax.experimental import pallas as pl
from jax.experimental.pallas import tpu as pltpu
from jax.experimental.pallas import tpu_sc as plsc

_NUM_EMB = 1000002
_D = 64
_B = 16384
_NW = 32
_BW = _B // _NW
_ROW_PITCH = (_NUM_EMB + 7) // 8 * 8


def _rsqrt16(x):
    i = lax.bitcast_convert_type(x, jnp.int32)
    i = jnp.int32(0x5F3759DF) - lax.shift_right_logical(i, 1)
    y = lax.bitcast_convert_type(i, jnp.float32)
    for _ in range(3):
        y = y * (jnp.float32(1.5) - jnp.float32(0.5) * x * y * y)
    return y


def _sc_body(table_t, nodes_hbm, out_hbm, idx_v, idxf, panel, rbuf, gsem):
    wid = lax.axis_index("s") * 2 + lax.axis_index("c")
    base = wid * _BW

    pltpu.sync_copy(nodes_hbm.at[pl.ds(base, _BW)], idx_v)

    def expand(d, _):
        row_off = d * _ROW_PITCH

        def inner(k, _):
            sl = pl.ds(k * 16, 16)
            idxf[d, sl] = idx_v[sl] + row_off
            return 0

        lax.fori_loop(0, _BW // 16, inner, 0)
        return 0

    lax.fori_loop(0, _D, expand, 0)

    flat = table_t.at[0]

    for d in range(_D):
        pltpu.make_async_copy(flat.at[idxf.at[d]], panel.at[d], gsem).start()

    def drain(d, _):
        pltpu.make_async_copy(flat.at[idxf.at[0]], panel.at[0], gsem).wait()
        return 0

    lax.fori_loop(0, _D, drain, 0)

    def norms(k, _):
        sl = pl.ds(k * 16, 16)

        def acc_d(d, acc):
            v = panel[d, sl]
            return acc + v * v

        acc = lax.fori_loop(0, _D, acc_d, jnp.zeros((16,), jnp.float32))
        rbuf[sl] = _rsqrt16(acc)
        return 0

    lax.fori_loop(0, _BW // 16, norms, 0)

    def scale(d, _):
        def inner(k, _):
            sl = pl.ds(k * 16, 16)
            panel[d, sl] = panel[d, sl] * rbuf[sl]
            return 0

        lax.fori_loop(0, _BW // 16, inner, 0)
        return 0

    lax.fori_loop(0, _D, scale, 0)

    pltpu.sync_copy(panel, out_hbm.at[:, pl.ds(base, _BW)])


@jax.jit
def _sc_call(table_t, nodes):
    mesh = plsc.VectorSubcoreMesh(core_axis_name="c", subcore_axis_name="s")
    return pl.kernel(
        _sc_body,
        out_type=jax.ShapeDtypeStruct((_D, _B), jnp.float32),
        mesh=mesh,
        compiler_params=pltpu.CompilerParams(
            needs_layout_passes=False, use_tc_tiling_on_sc=False
        ),
        scratch_types=[
            pltpu.VMEM((_BW,), jnp.int32),
            pltpu.VMEM((_D, _BW), jnp.int32),
            pltpu.VMEM((_D, _BW), jnp.float32),
            pltpu.VMEM((_BW,), jnp.float32),
            pltpu.SemaphoreType.DMA,
        ],
    )(table_t, nodes)


def kernel(nodes, table):
    return _sc_call(table.T, nodes)


# speedup vs baseline: 8.0557x; 8.0557x over previous
import functools

import jax
import jax.numpy as jnp
from jax import lax
from jax.experimental import pallas as pl
from jax.experimental.pallas import tpu as pltpu
from jax.experimental.pallas import tpu_sc as plsc

_NUM_EMB = 1000002
_D = 64
_B = 16384
_NW = 32
_BW = _B // _NW


def _rsqrt_scalar(x):
    i = lax.bitcast_convert_type(x, jnp.int32)
    i = jnp.int32(0x5F3759DF) - lax.shift_right_logical(i, 1)
    y = lax.bitcast_convert_type(i, jnp.float32)
    for _ in range(3):
        y = y * (jnp.float32(1.5) - jnp.float32(0.5) * x * y * y)
    return y


def _sc_body(table_hbm, nodes_hbm, out_hbm, idx4, panel, gsem):
    wid = lax.axis_index("s") * 2 + lax.axis_index("c")
    base = wid * _BW

    for j in range(4):
        pltpu.sync_copy(nodes_hbm.at[pl.ds(base + j * 128, 128)], idx4.at[j])
    for j in range(4):
        pltpu.make_async_copy(
            table_hbm.at[idx4.at[j]], panel.at[pl.ds(j * 128, 128)], gsem
        ).start()
    for j in range(4):
        pltpu.make_async_copy(
            table_hbm.at[idx4.at[0]], panel.at[pl.ds(0, 128)], gsem
        ).wait()

    def norm_one(i, _):
        def acc_k(k, acc):
            v = panel[i, pl.ds(k * 16, 16)]
            return acc + v * v

        acc = lax.fori_loop(0, _D // 16, acc_k, jnp.zeros((16,), jnp.float32))
        r = _rsqrt_scalar(jnp.sum(acc))

        def scale_k(k, _):
            sl = pl.ds(k * 16, 16)
            panel[i, sl] = panel[i, sl] * r
            return 0

        lax.fori_loop(0, _D // 16, scale_k, 0)
        return 0

    lax.fori_loop(0, _BW, norm_one, 0)

    pltpu.sync_copy(panel, out_hbm.at[pl.ds(base, _BW), :])


@jax.jit
def _sc_call(table, nodes):
    mesh = plsc.VectorSubcoreMesh(core_axis_name="c", subcore_axis_name="s")
    return pl.kernel(
        _sc_body,
        out_type=jax.ShapeDtypeStruct((_B, _D), jnp.float32),
        mesh=mesh,
        compiler_params=pltpu.CompilerParams(
            needs_layout_passes=False, use_tc_tiling_on_sc=False
        ),
        scratch_types=[
            pltpu.VMEM((4, 128), jnp.int32),
            pltpu.VMEM((_BW, _D), jnp.float32),
            pltpu.SemaphoreType.DMA,
        ],
    )(table, nodes)


def kernel(nodes, table):
    return _sc_call(table, nodes).T
